# baseline (speedup 1.0000x reference)
import functools

import jax
import jax.numpy as jnp
from jax.experimental import pallas as pl
from jax.experimental.pallas import tpu as pltpu

N_LATENTS = 64
BATCH = 16384
VOCAB = 10
VOCAB_PAD = 16
EMB_DIM = 50
EPS = 1e-5

CHUNK = 4096
GRID = BATCH // CHUNK


def _kernel(x_ref, xc_ref, emb_ref, gamma_ref, beta_ref, w_ref, b_ref,
            out1t_ref, out2t_ref, tbl_ref):
    i = pl.program_id(0)

    @pl.when(i == 0)
    def _compute_table():
        x = x_ref[...]
        emb = emb_ref[...]
        inv_b = 1.0 / BATCH
        mean = jnp.zeros((1, EMB_DIM), jnp.float32)
        counts = []
        for v in range(VOCAB):
            cnt = jnp.sum(jnp.where(x == v, 1.0, 0.0))
            counts.append(cnt)
            mean = mean + cnt * emb[v:v + 1, :]
        mean = mean * inv_b
        var = jnp.zeros((1, EMB_DIM), jnp.float32)
        for v in range(VOCAB):
            d = emb[v:v + 1, :] - mean
            var = var + counts[v] * (d * d)
        var = var * inv_b
        rstd = jax.lax.rsqrt(var + EPS)
        r = jnp.maximum((emb - mean) * rstd * gamma_ref[...] + beta_ref[...],
                        0.0)
        y = jax.lax.dot_general(r, w_ref[...], (((1,), (1,)), ((), ())),
                                preferred_element_type=jnp.float32)
        tbl_ref[...] = jnp.zeros((VOCAB_PAD, 2 * N_LATENTS), jnp.float32)
        tbl_ref[:VOCAB, :] = y + b_ref[...]

    xc = xc_ref[0]
    iota_v = jax.lax.broadcasted_iota(jnp.int32, (VOCAB_PAD, CHUNK), 0)
    onehot_t = jnp.where(xc == iota_v, 1.0, 0.0)
    yt = jax.lax.dot_general(tbl_ref[...], onehot_t, (((0,), (0,)), ((), ())),
                             preferred_element_type=jnp.float32)
    out1t_ref[...] = yt[:N_LATENTS, :]
    out2t_ref[...] = yt[N_LATENTS:, :]


@functools.partial(jax.jit, static_argnames=("interpret",))
def kernel(x, emb, gamma, beta, W, b, interpret=False):
    x_idx = x.astype(jnp.int32)
    x_mat = x_idx.reshape(128, 128)
    x3 = x_idx.reshape(GRID, 1, CHUNK)
    gamma1 = gamma.reshape(1, EMB_DIM)
    beta1 = beta.reshape(1, EMB_DIM)
    b1 = b.reshape(1, 2 * N_LATENTS)

    out1t, out2t = pl.pallas_call(
        _kernel,
        grid=(GRID,),
        in_specs=[
            pl.BlockSpec((128, 128), lambda i: (0, 0)),
            pl.BlockSpec((1, 1, CHUNK), lambda i: (i, 0, 0)),
            pl.BlockSpec((VOCAB, EMB_DIM), lambda i: (0, 0)),
            pl.BlockSpec((1, EMB_DIM), lambda i: (0, 0)),
            pl.BlockSpec((1, EMB_DIM), lambda i: (0, 0)),
            pl.BlockSpec((2 * N_LATENTS, EMB_DIM), lambda i: (0, 0)),
            pl.BlockSpec((1, 2 * N_LATENTS), lambda i: (0, 0)),
        ],
        out_specs=[
            pl.BlockSpec((N_LATENTS, CHUNK), lambda i: (0, i)),
            pl.BlockSpec((N_LATENTS, CHUNK), lambda i: (0, i)),
        ],
        out_shape=[
            jax.ShapeDtypeStruct((N_LATENTS, BATCH), jnp.float32),
            jax.ShapeDtypeStruct((N_LATENTS, BATCH), jnp.float32),
        ],
        scratch_shapes=[pltpu.VMEM((VOCAB_PAD, 2 * N_LATENTS), jnp.float32)],
        interpret=interpret,
    )(x_mat, x3, emb, gamma1, beta1, W, b1)
    return (out1t.T, out2t.T)


# speedup vs baseline: 1.0771x; 1.0771x over previous
import functools

import jax
import jax.numpy as jnp
from jax.experimental import pallas as pl
from jax.experimental.pallas import tpu as pltpu

N_LATENTS = 64
BATCH = 16384
VOCAB = 10
VOCAB_PAD = 16
EMB_DIM = 50
EPS = 1e-5

CHUNK = 8192
GRID = BATCH // CHUNK


def _kernel(x_ref, xc_ref, emb_ref, gamma_ref, beta_ref, w_ref, b_ref,
            out1t_ref, out2t_ref, tbl_ref):
    i = pl.program_id(0)

    @pl.when(i == 0)
    def _compute_table():
        x = x_ref[...]
        emb = emb_ref[...]
        inv_b = 1.0 / BATCH
        mean = jnp.zeros((1, EMB_DIM), jnp.float32)
        counts = []
        for v in range(VOCAB):
            cnt = jnp.sum(jnp.where(x == v, 1.0, 0.0))
            counts.append(cnt)
            mean = mean + cnt * emb[v:v + 1, :]
        mean = mean * inv_b
        var = jnp.zeros((1, EMB_DIM), jnp.float32)
        for v in range(VOCAB):
            d = emb[v:v + 1, :] - mean
            var = var + counts[v] * (d * d)
        var = var * inv_b
        rstd = jax.lax.rsqrt(var + EPS)
        r = jnp.maximum((emb - mean) * rstd * gamma_ref[...] + beta_ref[...],
                        0.0)
        y = jax.lax.dot_general(r, w_ref[...], (((1,), (1,)), ((), ())),
                                preferred_element_type=jnp.float32)
        tbl_ref[...] = jnp.zeros((VOCAB_PAD, 2 * N_LATENTS), jnp.float32)
        tbl_ref[:VOCAB, :] = y + b_ref[...]

    xc = xc_ref[0]
    iota_v = jax.lax.broadcasted_iota(jnp.int32, (VOCAB_PAD, CHUNK), 0)
    onehot_t = jnp.where(xc == iota_v, 1.0, 0.0)
    yt = jax.lax.dot_general(tbl_ref[...], onehot_t, (((0,), (0,)), ((), ())),
                             preferred_element_type=jnp.float32)
    out1t_ref[...] = yt[:N_LATENTS, :]
    out2t_ref[...] = yt[N_LATENTS:, :]


@functools.partial(jax.jit, static_argnames=("interpret",))
def kernel(x, emb, gamma, beta, W, b, interpret=False):
    x_idx = x.astype(jnp.int32)
    x_mat = x_idx.reshape(128, 128)
    x3 = x_idx.reshape(GRID, 1, CHUNK)
    gamma1 = gamma.reshape(1, EMB_DIM)
    beta1 = beta.reshape(1, EMB_DIM)
    b1 = b.reshape(1, 2 * N_LATENTS)

    out1t, out2t = pl.pallas_call(
        _kernel,
        grid=(GRID,),
        in_specs=[
            pl.BlockSpec((128, 128), lambda i: (0, 0)),
            pl.BlockSpec((1, 1, CHUNK), lambda i: (i, 0, 0)),
            pl.BlockSpec((VOCAB, EMB_DIM), lambda i: (0, 0)),
            pl.BlockSpec((1, EMB_DIM), lambda i: (0, 0)),
            pl.BlockSpec((1, EMB_DIM), lambda i: (0, 0)),
            pl.BlockSpec((2 * N_LATENTS, EMB_DIM), lambda i: (0, 0)),
            pl.BlockSpec((1, 2 * N_LATENTS), lambda i: (0, 0)),
        ],
        out_specs=[
            pl.BlockSpec((N_LATENTS, CHUNK), lambda i: (0, i)),
            pl.BlockSpec((N_LATENTS, CHUNK), lambda i: (0, i)),
        ],
        out_shape=[
            jax.ShapeDtypeStruct((N_LATENTS, BATCH), jnp.float32),
            jax.ShapeDtypeStruct((N_LATENTS, BATCH), jnp.float32),
        ],
        scratch_shapes=[pltpu.VMEM((VOCAB_PAD, 2 * N_LATENTS), jnp.float32)],
        interpret=interpret,
    )(x_mat, x3, emb, gamma1, beta1, W, b1)
    return (out1t.T, out2t.T)
